# Initial kernel scaffold; baseline (speedup 1.0000x reference)
import jax
import jax.numpy as jnp
from jax.experimental import pallas as pl


def kernel(inputs, adj, W, a):
    raise NotImplementedError("write your pallas kernel here")


# speedup vs baseline: 6.6485x; 6.6485x over previous
import functools

import jax
import jax.numpy as jnp
from jax import lax
from jax.experimental import pallas as pl
from jax.experimental.pallas import tpu as pltpu
from jax.experimental.pallas import tpu_sc as plsc


def kernel(inputs, adj, W, a):
    Nn, D = inputs.shape
    E = adj.shape[1]
    NC, NS = 2, 16
    NW = NC * NS
    EW = E // NW
    CH = 80
    NCHUNK = EW // CH
    NP = 10240
    STRIPE = NP // NS
    FB = NP // NW
    CHF = 80

    b = (a.reshape(2, D) * W[None, :]).astype(jnp.float32)
    src = adj[0].astype(jnp.int32)
    dst = adj[1].astype(jnp.int32)

    def s_body(b_ref, x_ref, o_ref):
        o_ref[...] = lax.dot_general(
            b_ref[...], x_ref[...], (((1,), (1,)), ((), ())),
            preferred_element_type=jnp.float32)

    s = pl.pallas_call(
        s_body,
        out_shape=jax.ShapeDtypeStruct((2, Nn), jnp.float32),
    )(b, inputs)

    mesh = plsc.VectorSubcoreMesh(core_axis_name="c", subcore_axis_name="s")

    @functools.partial(
        pl.kernel,
        out_type=[jax.ShapeDtypeStruct((NC, NP, D), jnp.float32),
                  jax.ShapeDtypeStruct((NC * NP,), jnp.float32)],
        mesh=mesh,
        compiler_params=pltpu.CompilerParams(needs_layout_passes=False),
        scratch_types=[
            pltpu.VMEM((Nn,), jnp.float32),
            pltpu.VMEM((Nn,), jnp.float32),
            pltpu.VMEM((CH,), jnp.int32),
            pltpu.VMEM((CH,), jnp.int32),
            pltpu.VMEM((CH, D), jnp.float32),
            pltpu.VMEM((CH, D), jnp.float32),
            pltpu.VMEM((CH,), jnp.float32),
            pltpu.VMEM_SHARED((NP, D), jnp.float32),
            pltpu.VMEM_SHARED((NP,), jnp.float32),
            pltpu.SemaphoreType.DMA,
        ],
    )
    def edge_kernel(x_hbm, src_hbm, dst_hbm, s_hbm, part_hbm, rs_hbm,
                    s1_v, s2_v, srcb, dstb, rows, wrow, wbuf, accs, rsacc,
                    sem):
        cid = lax.axis_index("c")
        sid = lax.axis_index("s")
        wid = cid * NS + sid

        zero16 = jnp.zeros((16,), jnp.float32)
        for j in range(CH):
            for k in range(D // 16):
                wrow[j, pl.ds(k * 16, 16)] = zero16
        for g in range(CH // 16):
            wbuf[pl.ds(g * 16, 16)] = zero16

        r0 = sid * STRIPE
        for k in range(STRIPE // CH):
            pltpu.sync_copy(wrow, accs.at[pl.ds(r0 + k * CH, CH)])
            pltpu.sync_copy(wbuf, rsacc.at[pl.ds(r0 + k * CH, CH)])

        pltpu.sync_copy(s_hbm.at[0], s1_v)
        pltpu.sync_copy(s_hbm.at[1], s2_v)
        plsc.subcore_barrier()

        ebase = wid * EW

        def chunk_body(i, carry):
            base = ebase + i * CH
            pltpu.sync_copy(src_hbm.at[pl.ds(base, CH)], srcb)
            pltpu.sync_copy(dst_hbm.at[pl.ds(base, CH)], dstb)
            pltpu.async_copy(x_hbm.at[dstb], rows, sem).wait()
            for g in range(CH // 16):
                s16 = srcb[pl.ds(g * 16, 16)]
                d16 = dstb[pl.ds(g * 16, 16)]
                t = plsc.load_gather(s1_v, [s16]) + plsc.load_gather(s2_v, [d16])
                t = jnp.where(t >= 0.0, t, 0.2 * t)
                w16 = jnp.exp(t)
                wbuf[pl.ds(g * 16, 16)] = w16
                for j2 in range(16):
                    ws = w16[j2]
                    r = g * 16 + j2
                    for k in range(D // 16):
                        sl = pl.ds(k * 16, 16)
                        wrow[r, sl] = rows[r, sl] * ws
            pltpu.sync_copy(wrow, accs.at[srcb], add=True)
            pltpu.sync_copy(wbuf, rsacc.at[srcb], add=True)
            return carry

        lax.fori_loop(0, NCHUNK, chunk_body, 0)

        plsc.subcore_barrier()
        pltpu.sync_copy(accs.at[pl.ds(r0, STRIPE)],
                        part_hbm.at[cid, pl.ds(r0, STRIPE)])
        pltpu.sync_copy(rsacc.at[pl.ds(r0, STRIPE)],
                        rs_hbm.at[pl.ds(cid * NP + r0, STRIPE)])

    part, rs = edge_kernel(inputs, src, dst, s)

    @functools.partial(
        pl.kernel,
        out_type=jax.ShapeDtypeStruct((Nn, D), jnp.float32),
        mesh=mesh,
        compiler_params=pltpu.CompilerParams(needs_layout_passes=False),
        scratch_types=[
            pltpu.VMEM((D,), jnp.float32),
            pltpu.VMEM((CHF, D), jnp.float32),
            pltpu.VMEM((CHF, D), jnp.float32),
            pltpu.VMEM((CHF,), jnp.float32),
            pltpu.VMEM((CHF,), jnp.float32),
            pltpu.VMEM((CHF, D), jnp.float32),
        ],
    )
    def fin_kernel(part_hbm, rs_hbm, w_hbm, out_hbm, wv, p0, p1, rs0, rs1,
                   outv):
        cid = lax.axis_index("c")
        sid = lax.axis_index("s")
        wid = cid * NS + sid
        pltpu.sync_copy(w_hbm, wv)
        base = wid * FB
        nch = jnp.maximum(0, jnp.minimum(FB // CHF, (Nn - base) // CHF))

        def chunk_body(k4, carry):
            r0 = base + k4 * CHF
            pltpu.sync_copy(part_hbm.at[0, pl.ds(r0, CHF)], p0)
            pltpu.sync_copy(part_hbm.at[1, pl.ds(r0, CHF)], p1)
            pltpu.sync_copy(rs_hbm.at[pl.ds(r0, CHF)], rs0)
            pltpu.sync_copy(rs_hbm.at[pl.ds(NP + r0, CHF)], rs1)
            for g in range(CHF // 16):
                den16 = rs0[pl.ds(g * 16, 16)] + rs1[pl.ds(g * 16, 16)]
                rden16 = 1.0 / den16
                for j2 in range(16):
                    rden = rden16[j2]
                    r = g * 16 + j2
                    for k in range(D // 16):
                        sl = pl.ds(k * 16, 16)
                        v = (p0[r, sl] + p1[r, sl]) * wv[sl] * rden
                        outv[r, sl] = jnp.where(v > 0.0, v, jnp.exp(v) - 1.0)
            pltpu.sync_copy(outv, out_hbm.at[pl.ds(r0, CHF)])
            return carry

        lax.fori_loop(0, nch, chunk_body, 0)

    return fin_kernel(part, rs, W)


# speedup vs baseline: 10.9873x; 1.6526x over previous
import functools

import jax
import jax.numpy as jnp
from jax import lax
from jax.experimental import pallas as pl
from jax.experimental.pallas import tpu as pltpu
from jax.experimental.pallas import tpu_sc as plsc


def kernel(inputs, adj, W, a):
    Nn, D = inputs.shape
    E = adj.shape[1]
    NC, NS = 2, 16
    NW = NC * NS
    EW = E // NW
    CH = 80
    NCHUNK = EW // CH
    NP = 10240
    STRIPE = NP // NS
    FB = NP // NW
    CHF = 80

    b = (a.reshape(2, D) * W[None, :]).astype(jnp.float32)
    src = adj[0].astype(jnp.int32)
    dst = adj[1].astype(jnp.int32)

    def s_body(b_ref, x_ref, o_ref):
        o_ref[...] = lax.dot_general(
            b_ref[...], x_ref[...], (((1,), (1,)), ((), ())),
            preferred_element_type=jnp.float32)

    s = pl.pallas_call(
        s_body,
        out_shape=jax.ShapeDtypeStruct((2, Nn), jnp.float32),
    )(b, inputs)

    mesh = plsc.VectorSubcoreMesh(core_axis_name="c", subcore_axis_name="s")

    @functools.partial(
        pl.kernel,
        out_type=[jax.ShapeDtypeStruct((NC, NP, D), jnp.float32),
                  jax.ShapeDtypeStruct((NC * NP,), jnp.float32)],
        mesh=mesh,
        compiler_params=pltpu.CompilerParams(needs_layout_passes=False),
        scratch_types=[
            pltpu.VMEM((CH,), jnp.int32),
            pltpu.VMEM((CH,), jnp.int32),
            pltpu.VMEM((CH,), jnp.int32),
            pltpu.VMEM((CH,), jnp.int32),
            pltpu.VMEM((CH,), jnp.int32),
            pltpu.VMEM((CH,), jnp.int32),
            pltpu.VMEM((CH,), jnp.float32),
            pltpu.VMEM((CH,), jnp.float32),
            pltpu.VMEM((CH,), jnp.float32),
            pltpu.VMEM((CH,), jnp.float32),
            pltpu.VMEM((CH, D), jnp.float32),
            pltpu.VMEM((CH, D), jnp.float32),
            pltpu.VMEM((CH, D), jnp.float32),
            pltpu.VMEM((CH, D), jnp.float32),
            pltpu.VMEM((CH,), jnp.float32),
            pltpu.VMEM((CH,), jnp.float32),
            pltpu.VMEM_SHARED((Nn,), jnp.float32),
            pltpu.VMEM_SHARED((Nn,), jnp.float32),
            pltpu.VMEM_SHARED((NP, D), jnp.float32),
            pltpu.VMEM_SHARED((NP,), jnp.float32),
            pltpu.SemaphoreType.DMA,
            pltpu.SemaphoreType.DMA,
            pltpu.SemaphoreType.DMA,
            pltpu.SemaphoreType.DMA,
            pltpu.SemaphoreType.DMA,
            pltpu.SemaphoreType.DMA,
            pltpu.SemaphoreType.DMA,
            pltpu.SemaphoreType.DMA,
        ],
    )
    def edge_kernel(x_hbm, src_hbm, dst_hbm, s_hbm, part_hbm, rs_hbm,
                    srcbA, srcbB, dstbA, dstbB, scbA, scbB,
                    svA, svB, dvA, dvB, rowsA, rowsB, wrowA, wrowB,
                    wbufA, wbufB, s1_sp, s2_sp, accs, rsacc,
                    isA, isB, gsA, gsB, ssA, ssB, rsA, rsB):
        cid = lax.axis_index("c")
        sid = lax.axis_index("s")
        wid = cid * NS + sid

        srcb_ = (srcbA, srcbB)
        dstb_ = (dstbA, dstbB)
        scb_ = (scbA, scbB)
        sv_ = (svA, svB)
        dv_ = (dvA, dvB)
        rows_ = (rowsA, rowsB)
        wrow_ = (wrowA, wrowB)
        wbuf_ = (wbufA, wbufB)
        is_ = (isA, isB)
        gs_ = (gsA, gsB)
        ss_ = (ssA, ssB)
        rs_ = (rsA, rsB)

        zero16 = jnp.zeros((16,), jnp.float32)
        for j in range(CH):
            for k in range(D // 16):
                wrowA[j, pl.ds(k * 16, 16)] = zero16
        for g in range(CH // 16):
            wbufA[pl.ds(g * 16, 16)] = zero16

        r0 = sid * STRIPE
        for k in range(STRIPE // CH):
            pltpu.sync_copy(wrowA, accs.at[pl.ds(r0 + k * CH, CH)])
            pltpu.sync_copy(wbufA, rsacc.at[pl.ds(r0 + k * CH, CH)])

        @pl.when(sid == 0)
        def _():
            pltpu.sync_copy(s_hbm.at[0], s1_sp)
            pltpu.sync_copy(s_hbm.at[1], s2_sp)

        plsc.subcore_barrier()

        ebase = wid * EW

        def issue_idx(X, i):
            base = ebase + i * CH
            pltpu.async_copy(src_hbm.at[pl.ds(base, CH)], srcb_[X], is_[X])
            pltpu.async_copy(dst_hbm.at[pl.ds(base, CH)], dstb_[X], is_[X])

        def wait_idx(X):
            pltpu.make_async_copy(src_hbm.at[pl.ds(0, CH)], srcb_[X],
                                  is_[X]).wait()
            pltpu.make_async_copy(dst_hbm.at[pl.ds(0, CH)], dstb_[X],
                                  is_[X]).wait()

        def issue_g(X):
            pltpu.async_copy(x_hbm.at[dstb_[X]], rows_[X], gs_[X])

        def wait_g(X):
            pltpu.make_async_copy(x_hbm.at[pl.ds(0, CH)], rows_[X],
                                  gs_[X]).wait()

        def issue_s(X):
            pltpu.async_copy(wrow_[X], accs.at[scb_[X]], ss_[X], add=True)
            pltpu.async_copy(wbuf_[X], rsacc.at[scb_[X]], rs_[X], add=True)

        def wait_s(X):
            pltpu.make_async_copy(wrow_[X], accs.at[pl.ds(0, CH)],
                                  ss_[X]).wait()
            pltpu.make_async_copy(wbuf_[X], rsacc.at[pl.ds(0, CH)],
                                  rs_[X]).wait()

        def stage_sv(X):
            for g in range(CH // 16):
                scb_[X][pl.ds(g * 16, 16)] = srcb_[X][pl.ds(g * 16, 16)]
            pltpu.sync_copy(s1_sp.at[srcb_[X]], sv_[X])
            pltpu.sync_copy(s2_sp.at[dstb_[X]], dv_[X])

        def compute(X):
            for g in range(CH // 16):
                t = sv_[X][pl.ds(g * 16, 16)] + dv_[X][pl.ds(g * 16, 16)]
                t = jnp.where(t >= 0.0, t, 0.2 * t)
                w16 = jnp.exp(t)
                wbuf_[X][pl.ds(g * 16, 16)] = w16
                for j2 in range(16):
                    ws = w16[j2]
                    r = g * 16 + j2
                    for k in range(D // 16):
                        sl = pl.ds(k * 16, 16)
                        wrow_[X][r, sl] = rows_[X][r, sl] * ws

        issue_idx(0, 0)
        issue_idx(1, 1)
        wait_idx(0)
        issue_g(0)

        def pair_body(p, carry):
            i0 = 2 * p
            wait_g(0)
            wait_idx(1)
            issue_g(1)

            @pl.when(p > 0)
            def _():
                wait_s(0)

            stage_sv(0)
            issue_idx(0, i0 + 2)
            compute(0)
            issue_s(0)

            wait_g(1)
            wait_idx(0)
            issue_g(0)

            @pl.when(p > 0)
            def _():
                wait_s(1)

            stage_sv(1)
            issue_idx(1, jnp.minimum(i0 + 3, NCHUNK - 1))
            compute(1)
            issue_s(1)
            return carry

        lax.fori_loop(0, (NCHUNK - 1) // 2, pair_body, 0)

        wait_g(0)
        wait_s(0)
        stage_sv(0)
        compute(0)
        issue_s(0)
        wait_idx(1)
        wait_s(0)
        wait_s(1)

        plsc.subcore_barrier()
        pltpu.sync_copy(accs.at[pl.ds(r0, STRIPE)],
                        part_hbm.at[cid, pl.ds(r0, STRIPE)])
        pltpu.sync_copy(rsacc.at[pl.ds(r0, STRIPE)],
                        rs_hbm.at[pl.ds(cid * NP + r0, STRIPE)])

    part, rs = edge_kernel(inputs, src, dst, s)

    @functools.partial(
        pl.kernel,
        out_type=jax.ShapeDtypeStruct((Nn, D), jnp.float32),
        mesh=mesh,
        compiler_params=pltpu.CompilerParams(needs_layout_passes=False),
        scratch_types=[
            pltpu.VMEM((D,), jnp.float32),
            pltpu.VMEM((CHF, D), jnp.float32),
            pltpu.VMEM((CHF, D), jnp.float32),
            pltpu.VMEM((CHF,), jnp.float32),
            pltpu.VMEM((CHF,), jnp.float32),
            pltpu.VMEM((CHF, D), jnp.float32),
        ],
    )
    def fin_kernel(part_hbm, rs_hbm, w_hbm, out_hbm, wv, p0, p1, rs0, rs1,
                   outv):
        cid = lax.axis_index("c")
        sid = lax.axis_index("s")
        wid = cid * NS + sid
        pltpu.sync_copy(w_hbm, wv)
        base = wid * FB
        nch = jnp.maximum(0, jnp.minimum(FB // CHF, (Nn - base) // CHF))

        def chunk_body(k4, carry):
            r0 = base + k4 * CHF
            pltpu.sync_copy(part_hbm.at[0, pl.ds(r0, CHF)], p0)
            pltpu.sync_copy(part_hbm.at[1, pl.ds(r0, CHF)], p1)
            pltpu.sync_copy(rs_hbm.at[pl.ds(r0, CHF)], rs0)
            pltpu.sync_copy(rs_hbm.at[pl.ds(NP + r0, CHF)], rs1)
            for g in range(CHF // 16):
                den16 = rs0[pl.ds(g * 16, 16)] + rs1[pl.ds(g * 16, 16)]
                rden16 = 1.0 / den16
                for j2 in range(16):
                    rden = rden16[j2]
                    r = g * 16 + j2
                    for k in range(D // 16):
                        sl = pl.ds(k * 16, 16)
                        v = (p0[r, sl] + p1[r, sl]) * wv[sl] * rden
                        outv[r, sl] = jnp.where(v > 0.0, v, jnp.exp(v) - 1.0)
            pltpu.sync_copy(outv, out_hbm.at[pl.ds(r0, CHF)])
            return carry

        lax.fori_loop(0, nch, chunk_body, 0)

    return fin_kernel(part, rs, W)


# speedup vs baseline: 14.2962x; 1.3011x over previous
import functools

import jax
import jax.numpy as jnp
from jax import lax
from jax.experimental import pallas as pl
from jax.experimental.pallas import tpu as pltpu
from jax.experimental.pallas import tpu_sc as plsc


def kernel(inputs, adj, W, a):
    Nn, D = inputs.shape
    E = adj.shape[1]
    NC, NS = 2, 16
    NW = NC * NS
    EW = E // NW
    CH = 80
    NCHUNK = EW // CH
    NP = 10240
    STRIPE = NP // NS
    FB = NP // NW
    CHF = 80

    adjf = adj.astype(jnp.int32).reshape(2 * E)

    def s_body(a_ref, w_ref, x_ref, o_ref):
        bb = a_ref[...].reshape(2, D) * w_ref[...]
        o_ref[...] = lax.dot_general(
            bb, x_ref[...], (((1,), (1,)), ((), ())),
            preferred_element_type=jnp.float32)

    s = pl.pallas_call(
        s_body,
        out_shape=jax.ShapeDtypeStruct((2, Nn), jnp.float32),
    )(a, W.reshape(1, D), inputs)

    mesh = plsc.VectorSubcoreMesh(core_axis_name="c", subcore_axis_name="s")

    @functools.partial(
        pl.kernel,
        out_type=[jax.ShapeDtypeStruct((NC, NP, D), jnp.float32),
                  jax.ShapeDtypeStruct((NC * NP,), jnp.float32)],
        mesh=mesh,
        compiler_params=pltpu.CompilerParams(needs_layout_passes=False),
        scratch_types=[
            pltpu.VMEM((CH,), jnp.int32),
            pltpu.VMEM((CH,), jnp.int32),
            pltpu.VMEM((CH,), jnp.int32),
            pltpu.VMEM((CH,), jnp.int32),
            pltpu.VMEM((CH,), jnp.int32),
            pltpu.VMEM((CH,), jnp.int32),
            pltpu.VMEM((CH,), jnp.float32),
            pltpu.VMEM((CH,), jnp.float32),
            pltpu.VMEM((CH,), jnp.float32),
            pltpu.VMEM((CH,), jnp.float32),
            pltpu.VMEM((CH, D), jnp.float32),
            pltpu.VMEM((CH, D), jnp.float32),
            pltpu.VMEM((CH, D), jnp.float32),
            pltpu.VMEM((CH, D), jnp.float32),
            pltpu.VMEM((CH,), jnp.float32),
            pltpu.VMEM((CH,), jnp.float32),
            pltpu.VMEM_SHARED((Nn,), jnp.float32),
            pltpu.VMEM_SHARED((Nn,), jnp.float32),
            pltpu.VMEM_SHARED((NP, D), jnp.float32),
            pltpu.VMEM_SHARED((NP,), jnp.float32),
            pltpu.SemaphoreType.DMA,
            pltpu.SemaphoreType.DMA,
            pltpu.SemaphoreType.DMA,
            pltpu.SemaphoreType.DMA,
            pltpu.SemaphoreType.DMA,
            pltpu.SemaphoreType.DMA,
            pltpu.SemaphoreType.DMA,
            pltpu.SemaphoreType.DMA,
            pltpu.SemaphoreType.DMA,
            pltpu.SemaphoreType.DMA,
        ],
    )
    def edge_kernel(x_hbm, adj_hbm, s_hbm, part_hbm, rs_hbm,
                    srcbA, srcbB, dstbA, dstbB, scbA, scbB,
                    svA, svB, dvA, dvB, rowsA, rowsB, wrowA, wrowB,
                    wbufA, wbufB, s1_sp, s2_sp, accs, rsacc,
                    isA, isB, gsA, gsB, ssA, ssB, rsA, rsB, vsA, vsB):
        cid = lax.axis_index("c")
        sid = lax.axis_index("s")
        wid = cid * NS + sid

        srcb_ = (srcbA, srcbB)
        dstb_ = (dstbA, dstbB)
        scb_ = (scbA, scbB)
        sv_ = (svA, svB)
        dv_ = (dvA, dvB)
        rows_ = (rowsA, rowsB)
        wrow_ = (wrowA, wrowB)
        wbuf_ = (wbufA, wbufB)
        is_ = (isA, isB)
        gs_ = (gsA, gsB)
        ss_ = (ssA, ssB)
        rs_ = (rsA, rsB)
        vs_ = (vsA, vsB)

        zero16 = jnp.zeros((16,), jnp.float32)
        for j in range(CH):
            for k in range(D // 16):
                wrowA[j, pl.ds(k * 16, 16)] = zero16
        for g in range(CH // 16):
            wbufA[pl.ds(g * 16, 16)] = zero16

        r0 = sid * STRIPE
        for k in range(STRIPE // CH):
            pltpu.sync_copy(wrowA, accs.at[pl.ds(r0 + k * CH, CH)])
            pltpu.sync_copy(wbufA, rsacc.at[pl.ds(r0 + k * CH, CH)])

        @pl.when(sid == 0)
        def _():
            pltpu.sync_copy(s_hbm.at[0], s1_sp)
            pltpu.sync_copy(s_hbm.at[1], s2_sp)

        plsc.subcore_barrier()

        ebase = wid * EW

        def issue_idx(X, i):
            base = ebase + i * CH
            pltpu.async_copy(adj_hbm.at[pl.ds(base, CH)], srcb_[X], is_[X])
            pltpu.async_copy(adj_hbm.at[pl.ds(E + base, CH)], dstb_[X],
                             is_[X])

        def wait_idx(X):
            pltpu.make_async_copy(adj_hbm.at[pl.ds(0, CH)], srcb_[X],
                                  is_[X]).wait()
            pltpu.make_async_copy(adj_hbm.at[pl.ds(0, CH)], dstb_[X],
                                  is_[X]).wait()

        def issue_g(X):
            pltpu.async_copy(x_hbm.at[dstb_[X]], rows_[X], gs_[X])

        def wait_g(X):
            pltpu.make_async_copy(x_hbm.at[pl.ds(0, CH)], rows_[X],
                                  gs_[X]).wait()

        def issue_s(X):
            pltpu.async_copy(wrow_[X], accs.at[scb_[X]], ss_[X], add=True)
            pltpu.async_copy(wbuf_[X], rsacc.at[scb_[X]], rs_[X], add=True)

        def wait_s(X):
            pltpu.make_async_copy(wrow_[X], accs.at[pl.ds(0, CH)],
                                  ss_[X]).wait()
            pltpu.make_async_copy(wbuf_[X], rsacc.at[pl.ds(0, CH)],
                                  rs_[X]).wait()

        def issue_sv(X):
            pltpu.async_copy(s1_sp.at[srcb_[X]], sv_[X], vs_[X])
            pltpu.async_copy(s2_sp.at[dstb_[X]], dv_[X], vs_[X])

        def wait_sv(X):
            pltpu.make_async_copy(s1_sp.at[pl.ds(0, CH)], sv_[X],
                                  vs_[X]).wait()
            pltpu.make_async_copy(s2_sp.at[pl.ds(0, CH)], dv_[X],
                                  vs_[X]).wait()

        def copy_scb(X):
            for g in range(CH // 16):
                scb_[X][pl.ds(g * 16, 16)] = srcb_[X][pl.ds(g * 16, 16)]

        def compute(X):
            for g in range(CH // 16):
                t = sv_[X][pl.ds(g * 16, 16)] + dv_[X][pl.ds(g * 16, 16)]
                t = jnp.where(t >= 0.0, t, 0.2 * t)
                w16 = jnp.exp(t)
                wbuf_[X][pl.ds(g * 16, 16)] = w16
                for j2 in range(16):
                    ws = w16[j2]
                    r = g * 16 + j2
                    for k in range(D // 16):
                        sl = pl.ds(k * 16, 16)
                        wrow_[X][r, sl] = rows_[X][r, sl] * ws

        issue_idx(0, 0)
        issue_idx(1, 1)
        wait_idx(0)
        issue_sv(0)
        issue_g(0)

        def pair_body(p, carry):
            i0 = 2 * p
            wait_idx(1)
            issue_sv(1)
            issue_g(1)

            @pl.when(p > 0)
            def _():
                wait_s(0)

            copy_scb(0)
            issue_idx(0, i0 + 2)
            wait_sv(0)
            wait_g(0)
            compute(0)
            issue_s(0)

            wait_idx(0)
            issue_sv(0)
            issue_g(0)

            @pl.when(p > 0)
            def _():
                wait_s(1)

            copy_scb(1)
            issue_idx(1, jnp.minimum(i0 + 3, NCHUNK - 1))
            wait_sv(1)
            wait_g(1)
            compute(1)
            issue_s(1)
            return carry

        lax.fori_loop(0, (NCHUNK - 1) // 2, pair_body, 0)

        wait_g(0)
        wait_s(0)
        copy_scb(0)
        wait_sv(0)
        compute(0)
        issue_s(0)
        wait_idx(1)
        wait_s(0)
        wait_s(1)

        plsc.subcore_barrier()
        pltpu.sync_copy(accs.at[pl.ds(r0, STRIPE)],
                        part_hbm.at[cid, pl.ds(r0, STRIPE)])
        pltpu.sync_copy(rsacc.at[pl.ds(r0, STRIPE)],
                        rs_hbm.at[pl.ds(cid * NP + r0, STRIPE)])

    part, rs = edge_kernel(inputs, adjf, s)

    rs2 = rs.reshape(NC, NP, 1)
    RB = 2000

    def fin_body(p0_ref, p1_ref, r0_ref, r1_ref, w_ref, o_ref):
        num = p0_ref[0] + p1_ref[0]
        den = r0_ref[0] + r1_ref[0]
        v = num * w_ref[...] / den
        o_ref[...] = jnp.where(v > 0.0, v, jnp.exp(v) - 1.0)

    out = pl.pallas_call(
        fin_body,
        grid=(Nn // RB,),
        in_specs=[
            pl.BlockSpec((1, RB, D), lambda i: (0, i, 0)),
            pl.BlockSpec((1, RB, D), lambda i: (1, i, 0)),
            pl.BlockSpec((1, RB, 1), lambda i: (0, i, 0)),
            pl.BlockSpec((1, RB, 1), lambda i: (1, i, 0)),
            pl.BlockSpec((1, D), lambda i: (0, 0)),
        ],
        out_specs=pl.BlockSpec((RB, D), lambda i: (i, 0)),
        out_shape=jax.ShapeDtypeStruct((Nn, D), jnp.float32),
    )(part, part, rs2, rs2, W.reshape(1, D))

    return out
